# Initial kernel scaffold; baseline (speedup 1.0000x reference)
import jax
import jax.numpy as jnp
from jax.experimental import pallas as pl


def kernel(x, edge_index, W1, b1, W2, b2, prelu_a):
    raise NotImplementedError("write your pallas kernel here")


# speedup vs baseline: 4.6116x; 4.6116x over previous
import functools

import jax
import jax.numpy as jnp
from jax import lax
from jax.experimental import pallas as pl
from jax.experimental.pallas import tpu as pltpu
from jax.experimental.pallas import tpu_sc as plsc

N = 10000
E = 160000
P = 10240
C = 128
NC = 2
NS = 16
NW = NC * NS
EB = 128
EP = 163840
NB = EP // EB
NB_W = NB // NW
RPS = P // NS
R = 1024
GRID = P // R

_mesh = plsc.VectorSubcoreMesh(core_axis_name="c", subcore_axis_name="s")


@functools.partial(
    pl.kernel,
    out_type=jax.ShapeDtypeStruct((NC, P, 8), jnp.float32),
    mesh=_mesh,
    scratch_types=[
        pltpu.VMEM_SHARED((P, 8), jnp.float32),
        pltpu.VMEM((NB_W, EB), jnp.int32),
        pltpu.VMEM((EB, 8), jnp.float32),
    ],
)
def _sc_deg(dst_hbm, ones_hbm, zeros_hbm, out_hbm, acc, dstbuf, ones):
    cid = lax.axis_index("c")
    sid = lax.axis_index("s")
    w = cid * NS + sid
    pltpu.sync_copy(ones_hbm, ones)
    pltpu.sync_copy(zeros_hbm.at[pl.ds(sid * RPS, RPS)],
                    acc.at[pl.ds(sid * RPS, RPS)])
    pltpu.sync_copy(dst_hbm.at[pl.ds(w * NB_W, NB_W)], dstbuf)
    plsc.subcore_barrier()

    @pl.loop(0, NB_W)
    def _(j):
        pltpu.sync_copy(ones, acc.at[dstbuf.at[j]], add=True)

    plsc.subcore_barrier()

    @pl.when(sid == 0)
    def _():
        pltpu.sync_copy(acc, out_hbm.at[cid])


def _make_sc_agg(K):
    @functools.partial(
        pl.kernel,
        out_type=[jax.ShapeDtypeStruct((NC, P, C), jnp.float32)
                  for _ in range(K)],
        mesh=_mesh,
        scratch_types=[
            pltpu.VMEM_SHARED((P, C), jnp.float32),
            pltpu.VMEM((NB_W, EB), jnp.int32),
            pltpu.VMEM((NB_W, EB), jnp.int32),
            pltpu.VMEM((EB, C), jnp.float32),
            pltpu.SemaphoreType.DMA,
        ],
    )
    def agg(src_hbm, dst_hbm, zeros_hbm, *rest):
        ys = rest[:K]
        outs = rest[K:2 * K]
        acc, srcbuf, dstbuf, rows, sem = rest[2 * K:]
        cid = lax.axis_index("c")
        sid = lax.axis_index("s")
        w = cid * NS + sid
        pltpu.sync_copy(src_hbm.at[pl.ds(w * NB_W, NB_W)], srcbuf)
        pltpu.sync_copy(dst_hbm.at[pl.ds(w * NB_W, NB_W)], dstbuf)

        for k in range(K):
            @pl.when(cid == 0)
            def _(k=k):
                pltpu.sync_copy(ys[k].at[pl.ds(sid * RPS, RPS)],
                                acc.at[pl.ds(sid * RPS, RPS)])

            @pl.when(cid != 0)
            def _():
                pltpu.sync_copy(zeros_hbm.at[pl.ds(sid * RPS, RPS)],
                                acc.at[pl.ds(sid * RPS, RPS)])

            plsc.subcore_barrier()

            @pl.loop(0, NB_W)
            def _(j, k=k):
                pltpu.async_copy(ys[k].at[srcbuf.at[j]], rows, sem).wait()
                pltpu.sync_copy(rows, acc.at[dstbuf.at[j]], add=True)

            plsc.subcore_barrier()

            @pl.when(sid == 0)
            def _(k=k):
                pltpu.sync_copy(acc, outs[k].at[cid])

            if k < K - 1:
                plsc.subcore_barrier()

    return agg


_sc_agg2 = _make_sc_agg(2)
_sc_agg4 = _make_sc_agg(4)


def _prescale_body(deg_ref, x_ref, y0_ref, y1_ref, dinv_ref):
    dp = deg_ref[...]
    deg = dp[0, :, 0:1] + dp[1, :, 0:1] + 1.0
    dv = lax.rsqrt(deg)
    rid = jax.lax.broadcasted_iota(jnp.int32, (R, 1), 0) + pl.program_id(0) * R
    mask = rid < N
    dv = jnp.where(mask, dv, 0.0)
    xb = x_ref[...]
    y = jnp.where(mask, xb * dv, 0.0)
    y0_ref[...] = y[:, :C]
    y1_ref[...] = y[:, C:]
    dinv_ref[...] = jnp.broadcast_to(dv, (R, C))


def _tc_prescale(deg_parts, x):
    return pl.pallas_call(
        _prescale_body,
        grid=(GRID,),
        in_specs=[
            pl.BlockSpec((NC, R, 8), lambda r: (0, r, 0)),
            pl.BlockSpec((R, 2 * C), lambda r: (r, 0)),
        ],
        out_specs=[
            pl.BlockSpec((R, C), lambda r: (r, 0)),
            pl.BlockSpec((R, C), lambda r: (r, 0)),
            pl.BlockSpec((R, C), lambda r: (r, 0)),
        ],
        out_shape=[
            jax.ShapeDtypeStruct((P, C), jnp.float32),
            jax.ShapeDtypeStruct((P, C), jnp.float32),
            jax.ShapeDtypeStruct((P, C), jnp.float32),
        ],
    )(deg_parts, x)


def _make_mm(K, F_out, final):
    def body(*refs):
        zs = refs[:K]
        dinv_ref, w_ref, b_ref, a_ref = refs[K:K + 4]
        outs = refs[K + 4:]
        dv = dinv_ref[:, 0:1]
        acc = b_ref[...]
        for k in range(K):
            zk = (zs[k][0] + zs[k][1]) * dv
            acc = acc + jnp.dot(zk, w_ref[k * C:(k + 1) * C, :],
                                preferred_element_type=jnp.float32)
        a = a_ref[0, 0]
        h = jnp.where(acc >= 0, acc, a * acc)
        if final:
            outs[0][...] = h
        else:
            h = h * dv
            for k in range(F_out // C):
                outs[k][...] = h[:, k * C:(k + 1) * C]

    n_out = 1 if final else F_out // C
    out_rows = N if final else P
    out_cols = F_out if final else C

    def run(zparts, dinv, W, b, a):
        return pl.pallas_call(
            body,
            grid=(GRID,),
            in_specs=(
                [pl.BlockSpec((NC, R, C), lambda r: (0, r, 0))
                 for _ in range(K)]
                + [
                    pl.BlockSpec((R, C), lambda r: (r, 0)),
                    pl.BlockSpec((K * C, F_out), lambda r: (0, 0)),
                    pl.BlockSpec((1, F_out), lambda r: (0, 0)),
                    pl.BlockSpec(memory_space=pltpu.SMEM),
                ]
            ),
            out_specs=[
                pl.BlockSpec((R, out_cols), lambda r: (r, 0))
                for _ in range(n_out)
            ],
            out_shape=[
                jax.ShapeDtypeStruct((out_rows, out_cols), jnp.float32)
                for _ in range(n_out)
            ],
        )(*zparts, dinv, W, b, a)

    return run


_tc_mm1 = _make_mm(2, 512, final=False)
_tc_mm2 = _make_mm(4, 512, final=True)


def kernel(x, edge_index, W1, b1, W2, b2, prelu_a):
    pad = jnp.full((EP - E,), N, dtype=jnp.int32)
    src2d = jnp.concatenate([edge_index[0], pad]).reshape(NB, EB)
    dst2d = jnp.concatenate([edge_index[1], pad]).reshape(NB, EB)
    ones8 = jnp.ones((EB, 8), jnp.float32)
    zeros8 = jnp.zeros((P, 8), jnp.float32)
    zerosC = jnp.zeros((P, C), jnp.float32)
    a2d = prelu_a.reshape(1, 1)
    b1r = b1.reshape(1, 512)
    b2r = b2.reshape(1, 512)

    deg_parts = _sc_deg(dst2d, ones8, zeros8)
    y10, y11, dinv = _tc_prescale(deg_parts, x)
    z1 = _sc_agg2(src2d, dst2d, zerosC, y10, y11)
    y2 = _tc_mm1(z1, dinv, W1, b1r, a2d)
    z2 = _sc_agg4(src2d, dst2d, zerosC, *y2)
    (out,) = _tc_mm2(z2, dinv, W2, b2r, a2d)
    return out


# speedup vs baseline: 12.0943x; 2.6226x over previous
import functools

import jax
import jax.numpy as jnp
from jax import lax
from jax.experimental import pallas as pl
from jax.experimental.pallas import tpu as pltpu
from jax.experimental.pallas import tpu_sc as plsc

N = 10000
E = 160000
P = 10240
C = 128
NC = 2
NS = 16
NW = NC * NS
EB = 128
EP = 163840
NB = EP // EB
NB_W = NB // NW
RPS = P // NS
R = 1024
GRID = P // R

_mesh = plsc.VectorSubcoreMesh(core_axis_name="c", subcore_axis_name="s")


@functools.partial(
    pl.kernel,
    out_type=jax.ShapeDtypeStruct((NC, P, 8), jnp.float32),
    mesh=_mesh,
    scratch_types=[
        pltpu.VMEM_SHARED((P, 8), jnp.float32),
        pltpu.VMEM((NB_W, EB), jnp.int32),
        pltpu.VMEM((EB, 8), jnp.float32),
    ],
)
def _sc_deg(dst_hbm, ones_hbm, zeros_hbm, out_hbm, acc, dstbuf, ones):
    cid = lax.axis_index("c")
    sid = lax.axis_index("s")
    w = cid * NS + sid
    pltpu.sync_copy(ones_hbm, ones)
    pltpu.sync_copy(zeros_hbm.at[pl.ds(sid * RPS, RPS)],
                    acc.at[pl.ds(sid * RPS, RPS)])
    pltpu.sync_copy(dst_hbm.at[pl.ds(w * NB_W, NB_W)], dstbuf)
    plsc.subcore_barrier()

    @pl.loop(0, NB_W)
    def _(j):
        pltpu.sync_copy(ones, acc.at[dstbuf.at[j]], add=True)

    plsc.subcore_barrier()

    @pl.when(sid == 0)
    def _():
        pltpu.sync_copy(acc, out_hbm.at[cid])


def _make_sc_agg(K):
    @functools.partial(
        pl.kernel,
        out_type=[jax.ShapeDtypeStruct((NC, P, C), jnp.float32)
                  for _ in range(K)],
        mesh=_mesh,
        scratch_types=[
            pltpu.VMEM_SHARED((P, C), jnp.float32),
            pltpu.VMEM((NB_W, EB), jnp.int32),
            pltpu.VMEM((NB_W, EB), jnp.int32),
            pltpu.VMEM((EB, C), jnp.float32),
            pltpu.SemaphoreType.DMA,
        ],
    )
    def agg(src_hbm, dst_hbm, zeros_hbm, *rest):
        ys = rest[:K]
        outs = rest[K:2 * K]
        acc, srcbuf, dstbuf, rows, sem = rest[2 * K:]
        cid = lax.axis_index("c")
        sid = lax.axis_index("s")
        w = cid * NS + sid
        pltpu.sync_copy(src_hbm.at[pl.ds(w * NB_W, NB_W)], srcbuf)
        pltpu.sync_copy(dst_hbm.at[pl.ds(w * NB_W, NB_W)], dstbuf)

        for k in range(K):
            @pl.when(cid == 0)
            def _(k=k):
                pltpu.sync_copy(ys[k].at[pl.ds(sid * RPS, RPS)],
                                acc.at[pl.ds(sid * RPS, RPS)])

            @pl.when(cid != 0)
            def _():
                pltpu.sync_copy(zeros_hbm.at[pl.ds(sid * RPS, RPS)],
                                acc.at[pl.ds(sid * RPS, RPS)])

            plsc.subcore_barrier()

            @pl.loop(0, NB_W)
            def _(j, k=k):
                pltpu.async_copy(ys[k].at[srcbuf.at[j]], rows, sem).wait()
                pltpu.sync_copy(rows, acc.at[dstbuf.at[j]], add=True)

            plsc.subcore_barrier()

            @pl.when(sid == 0)
            def _(k=k):
                pltpu.sync_copy(acc, outs[k].at[cid])

            if k < K - 1:
                plsc.subcore_barrier()

    return agg


_sc_agg2 = _make_sc_agg(2)
_sc_agg4 = _make_sc_agg(4)


def _prescale_body(deg_ref, x_ref, y0_ref, y1_ref, dinv_ref):
    dp = deg_ref[...]
    deg = dp[0, :, 0:1] + dp[1, :, 0:1] + 1.0
    dv = lax.rsqrt(deg)
    rid = jax.lax.broadcasted_iota(jnp.int32, (R, 1), 0) + pl.program_id(0) * R
    mask = rid < N
    dv = jnp.where(mask, dv, 0.0)
    xb = x_ref[...]
    y = jnp.where(mask, xb * dv, 0.0)
    y0_ref[...] = y[:, :C]
    y1_ref[...] = y[:, C:]
    dinv_ref[...] = jnp.broadcast_to(dv, (R, C))


def _tc_prescale(deg_parts, x):
    return pl.pallas_call(
        _prescale_body,
        grid=(GRID,),
        in_specs=[
            pl.BlockSpec((NC, R, 8), lambda r: (0, r, 0)),
            pl.BlockSpec((R, 2 * C), lambda r: (r, 0)),
        ],
        out_specs=[
            pl.BlockSpec((R, C), lambda r: (r, 0)),
            pl.BlockSpec((R, C), lambda r: (r, 0)),
            pl.BlockSpec((R, C), lambda r: (r, 0)),
        ],
        out_shape=[
            jax.ShapeDtypeStruct((P, C), jnp.float32),
            jax.ShapeDtypeStruct((P, C), jnp.float32),
            jax.ShapeDtypeStruct((P, C), jnp.float32),
        ],
    )(deg_parts, x)


def _make_mm(K, F_out, final):
    def body(*refs):
        zs = refs[:K]
        dinv_ref, w_ref, b_ref, a_ref = refs[K:K + 4]
        outs = refs[K + 4:]
        dv = dinv_ref[:, 0:1]
        acc = b_ref[...]
        for k in range(K):
            zk = (zs[k][0] + zs[k][1]) * dv
            acc = acc + jnp.dot(zk, w_ref[k * C:(k + 1) * C, :],
                                preferred_element_type=jnp.float32)
        a = a_ref[0, 0]
        h = jnp.where(acc >= 0, acc, a * acc)
        if final:
            outs[0][...] = h
        else:
            h = h * dv
            for k in range(F_out // C):
                outs[k][...] = h[:, k * C:(k + 1) * C]

    n_out = 1 if final else F_out // C
    out_rows = N if final else P
    out_cols = F_out if final else C

    def run(zparts, dinv, W, b, a):
        return pl.pallas_call(
            body,
            grid=(GRID,),
            in_specs=(
                [pl.BlockSpec((NC, R, C), lambda r: (0, r, 0))
                 for _ in range(K)]
                + [
                    pl.BlockSpec((R, C), lambda r: (r, 0)),
                    pl.BlockSpec((K * C, F_out), lambda r: (0, 0)),
                    pl.BlockSpec((1, F_out), lambda r: (0, 0)),
                    pl.BlockSpec(memory_space=pltpu.SMEM),
                ]
            ),
            out_specs=[
                pl.BlockSpec((R, out_cols), lambda r: (r, 0))
                for _ in range(n_out)
            ],
            out_shape=[
                jax.ShapeDtypeStruct((out_rows, out_cols), jnp.float32)
                for _ in range(n_out)
            ],
        )(*zparts, dinv, W, b, a)

    return run


_tc_mm1 = _make_mm(2, 512, final=False)
_tc_mm2 = _make_mm(4, 512, final=True)


def kernel(x, edge_index, W1, b1, W2, b2, prelu_a):
    pad = N + (jnp.arange(EP - E, dtype=jnp.int32) % (P - N))
    src2d = jnp.concatenate([edge_index[0], pad]).reshape(NB, EB)
    dst2d = jnp.concatenate([edge_index[1], pad]).reshape(NB, EB)
    ones8 = jnp.ones((EB, 8), jnp.float32)
    zeros8 = jnp.zeros((P, 8), jnp.float32)
    zerosC = jnp.zeros((P, C), jnp.float32)
    a2d = prelu_a.reshape(1, 1)
    b1r = b1.reshape(1, 512)
    b2r = b2.reshape(1, 512)

    deg_parts = _sc_deg(dst2d, ones8, zeros8)
    y10, y11, dinv = _tc_prescale(deg_parts, x)
    z1 = _sc_agg2(src2d, dst2d, zerosC, y10, y11)
    y2 = _tc_mm1(z1, dinv, W1, b1r, a2d)
    z2 = _sc_agg4(src2d, dst2d, zerosC, *y2)
    (out,) = _tc_mm2(z2, dinv, W2, b2r, a2d)
    return out


# speedup vs baseline: 12.1877x; 1.0077x over previous
import functools

import jax
import jax.numpy as jnp
from jax import lax
from jax.experimental import pallas as pl
from jax.experimental.pallas import tpu as pltpu
from jax.experimental.pallas import tpu_sc as plsc

N = 10000
E = 160000
P = 10240
C = 128
NC = 2
NS = 16
NW = NC * NS
EB = 128
EP = 163840
NB = EP // EB
NB_W = NB // NW
RPS = P // NS
R = 1024
GRID = P // R

_mesh = plsc.VectorSubcoreMesh(core_axis_name="c", subcore_axis_name="s")


@functools.partial(
    pl.kernel,
    out_type=jax.ShapeDtypeStruct((NC, P, 8), jnp.float32),
    mesh=_mesh,
    scratch_types=[
        pltpu.VMEM_SHARED((P, 8), jnp.float32),
        pltpu.VMEM((NB_W, EB), jnp.int32),
        pltpu.VMEM((EB, 8), jnp.float32),
    ],
)
def _sc_deg(dst_hbm, ones_hbm, zeros_hbm, out_hbm, acc, dstbuf, ones):
    cid = lax.axis_index("c")
    sid = lax.axis_index("s")
    w = cid * NS + sid
    pltpu.sync_copy(ones_hbm, ones)
    pltpu.sync_copy(zeros_hbm.at[pl.ds(sid * RPS, RPS)],
                    acc.at[pl.ds(sid * RPS, RPS)])
    pltpu.sync_copy(dst_hbm.at[pl.ds(w * NB_W, NB_W)], dstbuf)
    plsc.subcore_barrier()

    @pl.loop(0, NB_W)
    def _(j):
        pltpu.sync_copy(ones, acc.at[dstbuf.at[j]], add=True)

    plsc.subcore_barrier()

    @pl.when(sid == 0)
    def _():
        pltpu.sync_copy(acc, out_hbm.at[cid])


def _make_sc_agg(K):
    @functools.partial(
        pl.kernel,
        out_type=[jax.ShapeDtypeStruct((NC, P, C), jnp.float32)
                  for _ in range(K)],
        mesh=_mesh,
        scratch_types=[
            pltpu.VMEM_SHARED((P, C), jnp.float32),
            pltpu.VMEM((NB_W, EB), jnp.int32),
            pltpu.VMEM((NB_W, EB), jnp.int32),
            pltpu.VMEM((EB, C), jnp.float32),
            pltpu.SemaphoreType.DMA,
        ],
    )
    def agg(src_hbm, dst_hbm, zeros_hbm, *rest):
        ys = rest[:K]
        outs = rest[K:2 * K]
        acc, srcbuf, dstbuf, rows, sem = rest[2 * K:]
        cid = lax.axis_index("c")
        sid = lax.axis_index("s")
        w = cid * NS + sid
        pltpu.sync_copy(src_hbm.at[pl.ds(w * NB_W, NB_W)], srcbuf)
        pltpu.sync_copy(dst_hbm.at[pl.ds(w * NB_W, NB_W)], dstbuf)

        for k in range(K):
            @pl.when(cid == 0)
            def _(k=k):
                pltpu.sync_copy(ys[k].at[pl.ds(sid * RPS, RPS)],
                                acc.at[pl.ds(sid * RPS, RPS)])

            @pl.when(cid != 0)
            def _():
                pltpu.sync_copy(zeros_hbm.at[pl.ds(sid * RPS, RPS)],
                                acc.at[pl.ds(sid * RPS, RPS)])

            plsc.subcore_barrier()

            @pl.loop(0, NB_W)
            def _(j, k=k):
                pltpu.async_copy(ys[k].at[srcbuf.at[j]], rows, sem).wait()
                pltpu.sync_copy(rows, acc.at[dstbuf.at[j]], add=True)

            plsc.subcore_barrier()

            @pl.when(sid == 0)
            def _(k=k):
                pltpu.sync_copy(acc, outs[k].at[cid])

            if k < K - 1:
                plsc.subcore_barrier()

    return agg


_sc_agg2 = _make_sc_agg(2)
_sc_agg4 = _make_sc_agg(4)


def _prescale_body(deg_ref, x_ref, y0_ref, y1_ref, dinv_ref):
    dp = deg_ref[...]
    deg = dp[0, :, 0:1] + dp[1, :, 0:1] + 1.0
    dv = lax.rsqrt(deg)
    rid = jax.lax.broadcasted_iota(jnp.int32, (R, 1), 0) + pl.program_id(0) * R
    mask = rid < N
    dv = jnp.where(mask, dv, 0.0)
    xb = x_ref[...]
    y = jnp.where(mask, xb * dv, 0.0)
    y0_ref[...] = y[:, :C]
    y1_ref[...] = y[:, C:]
    dinv_ref[...] = jnp.broadcast_to(dv, (R, C))


def _tc_prescale(deg_parts, x):
    return pl.pallas_call(
        _prescale_body,
        grid=(GRID,),
        in_specs=[
            pl.BlockSpec((NC, R, 8), lambda r: (0, r, 0)),
            pl.BlockSpec((R, 2 * C), lambda r: (r, 0)),
        ],
        out_specs=[
            pl.BlockSpec((R, C), lambda r: (r, 0)),
            pl.BlockSpec((R, C), lambda r: (r, 0)),
            pl.BlockSpec((R, C), lambda r: (r, 0)),
        ],
        out_shape=[
            jax.ShapeDtypeStruct((P, C), jnp.float32),
            jax.ShapeDtypeStruct((P, C), jnp.float32),
            jax.ShapeDtypeStruct((P, C), jnp.float32),
        ],
    )(deg_parts, x)


def _make_mm(K, F_out, final):
    def body(*refs):
        zs = refs[:K]
        dinv_ref, w_ref, b_ref, a_ref = refs[K:K + 4]
        outs = refs[K + 4:]
        dv = dinv_ref[:, 0:1]
        acc = b_ref[...]
        for k in range(K):
            zk = (zs[k][0] + zs[k][1]) * dv
            acc = acc + jnp.dot(zk, w_ref[k * C:(k + 1) * C, :],
                                preferred_element_type=jnp.float32)
        a = a_ref[0, 0]
        h = jnp.where(acc >= 0, acc, a * acc)
        if final:
            outs[0][...] = h
        else:
            h = h * dv
            for k in range(F_out // C):
                outs[k][...] = h[:, k * C:(k + 1) * C]

    n_out = 1 if final else F_out // C
    out_rows = N if final else P
    out_cols = F_out if final else C

    def run(zparts, dinv, W, b, a):
        return pl.pallas_call(
            body,
            grid=(GRID,),
            in_specs=(
                [pl.BlockSpec((NC, R, C), lambda r: (0, r, 0))
                 for _ in range(K)]
                + [
                    pl.BlockSpec((R, C), lambda r: (r, 0)),
                    pl.BlockSpec((K * C, F_out), lambda r: (0, 0)),
                    pl.BlockSpec((1, F_out), lambda r: (0, 0)),
                    pl.BlockSpec(memory_space=pltpu.SMEM),
                ]
            ),
            out_specs=[
                pl.BlockSpec((R, out_cols), lambda r: (r, 0))
                for _ in range(n_out)
            ],
            out_shape=[
                jax.ShapeDtypeStruct((out_rows, out_cols), jnp.float32)
                for _ in range(n_out)
            ],
        )(*zparts, dinv, W, b, a)

    return run


_tc_mm1 = _make_mm(2, 512, final=False)
_tc_mm2 = _make_mm(4, 512, final=True)


def kernel(x, edge_index, W1, b1, W2, b2, prelu_a):
    pad = N + (jnp.arange(EP - E, dtype=jnp.int32) % (P - N))
    src2d = jnp.concatenate([edge_index[0], pad]).reshape(NB, EB)
    dst2d = jnp.concatenate([edge_index[1], pad]).reshape(NB, EB)
    padN = jnp.full((EP - E,), N, dtype=jnp.int32)
    dst2d_deg = jnp.concatenate([edge_index[1], padN]).reshape(NB, EB)
    ones8 = jnp.ones((EB, 8), jnp.float32)
    zeros8 = jnp.zeros((P, 8), jnp.float32)
    zerosC = jnp.zeros((P, C), jnp.float32)
    a2d = prelu_a.reshape(1, 1)
    b1r = b1.reshape(1, 512)
    b2r = b2.reshape(1, 512)

    deg_parts = _sc_deg(dst2d_deg, ones8, zeros8)
    y10, y11, dinv = _tc_prescale(deg_parts, x)
    z1 = _sc_agg2(src2d, dst2d, zerosC, y10, y11)
    y2 = _tc_mm1(z1, dinv, W1, b1r, a2d)
    z2 = _sc_agg4(src2d, dst2d, zerosC, *y2)
    (out,) = _tc_mm2(z2, dinv, W2, b2r, a2d)
    return out


# speedup vs baseline: 12.2314x; 1.0036x over previous
import functools

import jax
import jax.numpy as jnp
from jax import lax
from jax.experimental import pallas as pl
from jax.experimental.pallas import tpu as pltpu
from jax.experimental.pallas import tpu_sc as plsc

N = 10000
E = 160000
P = 10240
C = 128
NC = 2
NS = 16
NW = NC * NS
EB = 125
NB = E // EB
NB_W = NB // NW
RPS = P // NS
R = 1024
GRID = P // R

_mesh = plsc.VectorSubcoreMesh(core_axis_name="c", subcore_axis_name="s")


@functools.partial(
    pl.kernel,
    out_type=jax.ShapeDtypeStruct((NC, P, 8), jnp.float32),
    mesh=_mesh,
    scratch_types=[
        pltpu.VMEM_SHARED((P, 8), jnp.float32),
        pltpu.VMEM((NB_W, EB), jnp.int32),
        pltpu.VMEM((EB, 8), jnp.float32),
    ],
)
def _sc_deg(dst_hbm, ones_hbm, zeros_hbm, out_hbm, acc, dstbuf, ones):
    cid = lax.axis_index("c")
    sid = lax.axis_index("s")
    w = cid * NS + sid
    pltpu.sync_copy(ones_hbm, ones)
    pltpu.sync_copy(zeros_hbm.at[pl.ds(sid * RPS, RPS)],
                    acc.at[pl.ds(sid * RPS, RPS)])
    pltpu.sync_copy(dst_hbm.at[pl.ds(w * NB_W, NB_W)], dstbuf)
    plsc.subcore_barrier()

    @pl.loop(0, NB_W)
    def _(j):
        pltpu.sync_copy(ones, acc.at[dstbuf.at[j]], add=True)

    plsc.subcore_barrier()

    @pl.when(sid == 0)
    def _():
        pltpu.sync_copy(acc, out_hbm.at[cid])


def _make_sc_agg(K):
    @functools.partial(
        pl.kernel,
        out_type=[jax.ShapeDtypeStruct((NC, P, C), jnp.float32)
                  for _ in range(K)],
        mesh=_mesh,
        scratch_types=[
            pltpu.VMEM_SHARED((P, C), jnp.float32),
            pltpu.VMEM((NB_W, EB), jnp.int32),
            pltpu.VMEM((NB_W, EB), jnp.int32),
            pltpu.VMEM((EB, C), jnp.float32),
            pltpu.SemaphoreType.DMA,
        ],
    )
    def agg(src_hbm, dst_hbm, zeros_hbm, *rest):
        ys = rest[:K]
        outs = rest[K:2 * K]
        acc, srcbuf, dstbuf, rows, sem = rest[2 * K:]
        cid = lax.axis_index("c")
        sid = lax.axis_index("s")
        w = cid * NS + sid
        pltpu.sync_copy(src_hbm.at[pl.ds(w * NB_W, NB_W)], srcbuf)
        pltpu.sync_copy(dst_hbm.at[pl.ds(w * NB_W, NB_W)], dstbuf)

        for k in range(K):
            @pl.when(cid == 0)
            def _(k=k):
                pltpu.sync_copy(ys[k].at[pl.ds(sid * RPS, RPS)],
                                acc.at[pl.ds(sid * RPS, RPS)])

            @pl.when(cid != 0)
            def _():
                pltpu.sync_copy(zeros_hbm.at[pl.ds(sid * RPS, RPS)],
                                acc.at[pl.ds(sid * RPS, RPS)])

            plsc.subcore_barrier()

            @pl.loop(0, NB_W)
            def _(j, k=k):
                pltpu.async_copy(ys[k].at[srcbuf.at[j]], rows, sem).wait()
                pltpu.sync_copy(rows, acc.at[dstbuf.at[j]], add=True)

            plsc.subcore_barrier()

            @pl.when(sid == 0)
            def _(k=k):
                pltpu.sync_copy(acc, outs[k].at[cid])

            if k < K - 1:
                plsc.subcore_barrier()

    return agg


_sc_agg2 = _make_sc_agg(2)
_sc_agg4 = _make_sc_agg(4)


def _prescale_body(deg_ref, x_ref, y0_ref, y1_ref, dinv_ref):
    dp = deg_ref[...]
    deg = dp[0, :, 0:1] + dp[1, :, 0:1] + 1.0
    dv = lax.rsqrt(deg)
    rid = jax.lax.broadcasted_iota(jnp.int32, (R, 1), 0) + pl.program_id(0) * R
    mask = rid < N
    dv = jnp.where(mask, dv, 0.0)
    xb = x_ref[...]
    y = jnp.where(mask, xb * dv, 0.0)
    y0_ref[...] = y[:, :C]
    y1_ref[...] = y[:, C:]
    dinv_ref[...] = jnp.broadcast_to(dv, (R, C))


def _tc_prescale(deg_parts, x):
    return pl.pallas_call(
        _prescale_body,
        grid=(GRID,),
        in_specs=[
            pl.BlockSpec((NC, R, 8), lambda r: (0, r, 0)),
            pl.BlockSpec((R, 2 * C), lambda r: (r, 0)),
        ],
        out_specs=[
            pl.BlockSpec((R, C), lambda r: (r, 0)),
            pl.BlockSpec((R, C), lambda r: (r, 0)),
            pl.BlockSpec((R, C), lambda r: (r, 0)),
        ],
        out_shape=[
            jax.ShapeDtypeStruct((P, C), jnp.float32),
            jax.ShapeDtypeStruct((P, C), jnp.float32),
            jax.ShapeDtypeStruct((P, C), jnp.float32),
        ],
    )(deg_parts, x)


def _make_mm(K, F_out, final):
    def body(*refs):
        zs = refs[:K]
        dinv_ref, w_ref, b_ref, a_ref = refs[K:K + 4]
        outs = refs[K + 4:]
        dv = dinv_ref[:, 0:1]
        acc = b_ref[...]
        for k in range(K):
            zk = (zs[k][0] + zs[k][1]) * dv
            acc = acc + jnp.dot(zk, w_ref[k * C:(k + 1) * C, :],
                                preferred_element_type=jnp.float32)
        a = a_ref[0, 0]
        h = jnp.where(acc >= 0, acc, a * acc)
        if final:
            outs[0][...] = h
        else:
            h = h * dv
            for k in range(F_out // C):
                outs[k][...] = h[:, k * C:(k + 1) * C]

    n_out = 1 if final else F_out // C
    out_rows = N if final else P
    out_cols = F_out if final else C

    def run(zparts, dinv, W, b, a):
        return pl.pallas_call(
            body,
            grid=(GRID,),
            in_specs=(
                [pl.BlockSpec((NC, R, C), lambda r: (0, r, 0))
                 for _ in range(K)]
                + [
                    pl.BlockSpec((R, C), lambda r: (r, 0)),
                    pl.BlockSpec((K * C, F_out), lambda r: (0, 0)),
                    pl.BlockSpec((1, F_out), lambda r: (0, 0)),
                    pl.BlockSpec(memory_space=pltpu.SMEM),
                ]
            ),
            out_specs=[
                pl.BlockSpec((R, out_cols), lambda r: (r, 0))
                for _ in range(n_out)
            ],
            out_shape=[
                jax.ShapeDtypeStruct((out_rows, out_cols), jnp.float32)
                for _ in range(n_out)
            ],
        )(*zparts, dinv, W, b, a)

    return run


_tc_mm1 = _make_mm(2, 512, final=False)
_tc_mm2 = _make_mm(4, 512, final=True)


def kernel(x, edge_index, W1, b1, W2, b2, prelu_a):
    src2d = edge_index[0].reshape(NB, EB)
    dst2d = edge_index[1].reshape(NB, EB)
    ones8 = jnp.ones((EB, 8), jnp.float32)
    zeros8 = jnp.zeros((P, 8), jnp.float32)
    zerosC = jnp.zeros((P, C), jnp.float32)
    a2d = prelu_a.reshape(1, 1)
    b1r = b1.reshape(1, 512)
    b2r = b2.reshape(1, 512)

    deg_parts = _sc_deg(dst2d, ones8, zeros8)
    y10, y11, dinv = _tc_prescale(deg_parts, x)
    z1 = _sc_agg2(src2d, dst2d, zerosC, y10, y11)
    y2 = _tc_mm1(z1, dinv, W1, b1r, a2d)
    z2 = _sc_agg4(src2d, dst2d, zerosC, *y2)
    (out,) = _tc_mm2(z2, dinv, W2, b2r, a2d)
    return out


# speedup vs baseline: 12.3422x; 1.0091x over previous
import functools

import jax
import jax.numpy as jnp
from jax import lax
from jax.experimental import pallas as pl
from jax.experimental.pallas import tpu as pltpu
from jax.experimental.pallas import tpu_sc as plsc

N = 10000
E = 160000
P = 10240
C = 128
NC = 2
NS = 16
NW = NC * NS
EB = 125
NB = E // EB
NB_W = NB // NW
RPS = P // NS
R = 1024
GRID = P // R

_mesh = plsc.VectorSubcoreMesh(core_axis_name="c", subcore_axis_name="s")


@functools.partial(
    pl.kernel,
    out_type=jax.ShapeDtypeStruct((NC, P, C), jnp.float32),
    mesh=_mesh,
    scratch_types=[
        pltpu.VMEM_SHARED((P, C), jnp.float32),
        pltpu.VMEM((NB_W, EB), jnp.int32),
        pltpu.VMEM((EB, C), jnp.float32),
    ],
)
def _sc_deg(dst_hbm, ones_hbm, zeros_hbm, out_hbm, acc, dstbuf, ones):
    cid = lax.axis_index("c")
    sid = lax.axis_index("s")
    w = cid * NS + sid
    pltpu.sync_copy(ones_hbm, ones)
    pltpu.sync_copy(zeros_hbm.at[pl.ds(sid * RPS, RPS)],
                    acc.at[pl.ds(sid * RPS, RPS)])
    pltpu.sync_copy(dst_hbm.at[pl.ds(w * NB_W, NB_W)], dstbuf)
    plsc.subcore_barrier()

    @pl.loop(0, NB_W)
    def _(j):
        pltpu.sync_copy(ones, acc.at[dstbuf.at[j]], add=True)

    plsc.subcore_barrier()

    @pl.when(sid == 0)
    def _():
        pltpu.sync_copy(acc, out_hbm.at[cid])


def _make_sc_agg(K):
    @functools.partial(
        pl.kernel,
        out_type=[jax.ShapeDtypeStruct((NC, P, C), jnp.float32)
                  for _ in range(K)],
        mesh=_mesh,
        scratch_types=[
            pltpu.VMEM_SHARED((P, C), jnp.float32),
            pltpu.VMEM((NB_W, EB), jnp.int32),
            pltpu.VMEM((NB_W, EB), jnp.int32),
            pltpu.VMEM((EB, C), jnp.float32),
            pltpu.SemaphoreType.DMA,
        ],
    )
    def agg(src_hbm, dst_hbm, zeros_hbm, *rest):
        ys = rest[:K]
        outs = rest[K:2 * K]
        acc, srcbuf, dstbuf, rows, sem = rest[2 * K:]
        cid = lax.axis_index("c")
        sid = lax.axis_index("s")
        w = cid * NS + sid
        pltpu.sync_copy(src_hbm.at[pl.ds(w * NB_W, NB_W)], srcbuf)
        pltpu.sync_copy(dst_hbm.at[pl.ds(w * NB_W, NB_W)], dstbuf)

        for k in range(K):
            @pl.when(cid == 0)
            def _(k=k):
                pltpu.sync_copy(ys[k].at[pl.ds(sid * RPS, RPS)],
                                acc.at[pl.ds(sid * RPS, RPS)])

            @pl.when(cid != 0)
            def _():
                pltpu.sync_copy(zeros_hbm.at[pl.ds(sid * RPS, RPS)],
                                acc.at[pl.ds(sid * RPS, RPS)])

            plsc.subcore_barrier()

            @pl.loop(0, NB_W)
            def _(j, k=k):
                pltpu.async_copy(ys[k].at[srcbuf.at[j]], rows, sem).wait()
                pltpu.sync_copy(rows, acc.at[dstbuf.at[j]], add=True)

            plsc.subcore_barrier()

            @pl.when(sid == 0)
            def _(k=k):
                pltpu.sync_copy(acc, outs[k].at[cid])

            if k < K - 1:
                plsc.subcore_barrier()

    return agg


_sc_agg2 = _make_sc_agg(2)
_sc_agg4 = _make_sc_agg(4)


def _prescale_body(deg_ref, x_ref, y0_ref, y1_ref, dinv_ref):
    dp = deg_ref[...]
    deg = dp[0, :, 0:1] + dp[1, :, 0:1] + 1.0
    dv = lax.rsqrt(deg)
    rid = jax.lax.broadcasted_iota(jnp.int32, (R, 1), 0) + pl.program_id(0) * R
    mask = rid < N
    dv = jnp.where(mask, dv, 0.0)
    xb = x_ref[...]
    y = jnp.where(mask, xb * dv, 0.0)
    y0_ref[...] = y[:, :C]
    y1_ref[...] = y[:, C:]
    dinv_ref[...] = jnp.broadcast_to(dv, (R, C))


def _tc_prescale(deg_parts, x):
    return pl.pallas_call(
        _prescale_body,
        grid=(GRID,),
        in_specs=[
            pl.BlockSpec((NC, R, C), lambda r: (0, r, 0)),
            pl.BlockSpec((R, 2 * C), lambda r: (r, 0)),
        ],
        out_specs=[
            pl.BlockSpec((R, C), lambda r: (r, 0)),
            pl.BlockSpec((R, C), lambda r: (r, 0)),
            pl.BlockSpec((R, C), lambda r: (r, 0)),
        ],
        out_shape=[
            jax.ShapeDtypeStruct((P, C), jnp.float32),
            jax.ShapeDtypeStruct((P, C), jnp.float32),
            jax.ShapeDtypeStruct((P, C), jnp.float32),
        ],
    )(deg_parts, x)


def _make_mm(K, F_out, final):
    def body(*refs):
        zs = refs[:K]
        dinv_ref, w_ref, b_ref, a_ref = refs[K:K + 4]
        outs = refs[K + 4:]
        dv = dinv_ref[:, 0:1]
        acc = b_ref[...]
        for k in range(K):
            zk = (zs[k][0] + zs[k][1]) * dv
            acc = acc + jnp.dot(zk, w_ref[k * C:(k + 1) * C, :],
                                preferred_element_type=jnp.float32)
        a = a_ref[0, 0]
        h = jnp.where(acc >= 0, acc, a * acc)
        if final:
            outs[0][...] = h
        else:
            h = h * dv
            for k in range(F_out // C):
                outs[k][...] = h[:, k * C:(k + 1) * C]

    n_out = 1 if final else F_out // C
    out_rows = N if final else P
    out_cols = F_out if final else C

    def run(zparts, dinv, W, b, a):
        return pl.pallas_call(
            body,
            grid=(GRID,),
            in_specs=(
                [pl.BlockSpec((NC, R, C), lambda r: (0, r, 0))
                 for _ in range(K)]
                + [
                    pl.BlockSpec((R, C), lambda r: (r, 0)),
                    pl.BlockSpec((K * C, F_out), lambda r: (0, 0)),
                    pl.BlockSpec((1, F_out), lambda r: (0, 0)),
                    pl.BlockSpec(memory_space=pltpu.SMEM),
                ]
            ),
            out_specs=[
                pl.BlockSpec((R, out_cols), lambda r: (r, 0))
                for _ in range(n_out)
            ],
            out_shape=[
                jax.ShapeDtypeStruct((out_rows, out_cols), jnp.float32)
                for _ in range(n_out)
            ],
        )(*zparts, dinv, W, b, a)

    return run


_tc_mm1 = _make_mm(2, 512, final=False)
_tc_mm2 = _make_mm(4, 512, final=True)


def kernel(x, edge_index, W1, b1, W2, b2, prelu_a):
    src2d = edge_index[0].reshape(NB, EB)
    dst2d = edge_index[1].reshape(NB, EB)
    onesC = jnp.ones((EB, C), jnp.float32)
    zerosC = jnp.zeros((P, C), jnp.float32)
    a2d = prelu_a.reshape(1, 1)
    b1r = b1.reshape(1, 512)
    b2r = b2.reshape(1, 512)

    deg_parts = _sc_deg(dst2d, onesC, zerosC)
    y10, y11, dinv = _tc_prescale(deg_parts, x)
    z1 = _sc_agg2(src2d, dst2d, zerosC, y10, y11)
    y2 = _tc_mm1(z1, dinv, W1, b1r, a2d)
    z2 = _sc_agg4(src2d, dst2d, zerosC, *y2)
    (out,) = _tc_mm2(z2, dinv, W2, b2r, a2d)
    return out


# speedup vs baseline: 17.3513x; 1.4059x over previous
import functools

import jax
import jax.numpy as jnp
from jax import lax
from jax.experimental import pallas as pl
from jax.experimental.pallas import tpu as pltpu
from jax.experimental.pallas import tpu_sc as plsc

N = 10000
E = 160000
P = 10240
C = 128
NC = 2
NS = 16
NW = NC * NS
EB = 125
NB = E // EB
NB_W = NB // NW
RPS = P // NS
R = 1024
GRID = P // R

_mesh = plsc.VectorSubcoreMesh(core_axis_name="c", subcore_axis_name="s")


@functools.partial(
    pl.kernel,
    out_type=jax.ShapeDtypeStruct((NC, P, C), jnp.float32),
    mesh=_mesh,
    scratch_types=[
        pltpu.VMEM_SHARED((P, C), jnp.float32),
        pltpu.VMEM((NB_W, EB), jnp.int32),
        pltpu.VMEM((EB, C), jnp.float32),
    ],
)
def _sc_deg(dst_hbm, ones_hbm, zeros_hbm, out_hbm, acc, dstbuf, ones):
    cid = lax.axis_index("c")
    sid = lax.axis_index("s")
    w = cid * NS + sid
    pltpu.sync_copy(ones_hbm, ones)
    pltpu.sync_copy(zeros_hbm.at[pl.ds(sid * RPS, RPS)],
                    acc.at[pl.ds(sid * RPS, RPS)])
    pltpu.sync_copy(dst_hbm.at[pl.ds(w * NB_W, NB_W)], dstbuf)
    plsc.subcore_barrier()

    @pl.loop(0, NB_W)
    def _(j):
        pltpu.sync_copy(ones, acc.at[dstbuf.at[j]], add=True)

    plsc.subcore_barrier()

    @pl.when(sid == 0)
    def _():
        pltpu.sync_copy(acc, out_hbm.at[cid])


def _make_sc_agg(K):
    @functools.partial(
        pl.kernel,
        out_type=[jax.ShapeDtypeStruct((NC, P, C), jnp.float32)
                  for _ in range(K)],
        mesh=_mesh,
        scratch_types=[
            pltpu.VMEM_SHARED((P, C), jnp.float32),
            pltpu.VMEM((NB_W, EB), jnp.int32),
            pltpu.VMEM((NB_W, EB), jnp.int32),
            pltpu.VMEM((EB, C), jnp.float32),
            pltpu.VMEM((EB, C), jnp.float32),
            pltpu.SemaphoreType.DMA,
            pltpu.SemaphoreType.DMA,
        ],
    )
    def agg(src_hbm, dst_hbm, zeros_hbm, *rest):
        ys = rest[:K]
        outs = rest[K:2 * K]
        acc, srcbuf, dstbuf, rows_a, rows_b, sem_a, sem_b = rest[2 * K:]
        cid = lax.axis_index("c")
        sid = lax.axis_index("s")
        w = cid * NS + sid
        pltpu.sync_copy(src_hbm.at[pl.ds(w * NB_W, NB_W)], srcbuf)
        pltpu.sync_copy(dst_hbm.at[pl.ds(w * NB_W, NB_W)], dstbuf)

        for k in range(K):
            @pl.when(cid == 0)
            def _(k=k):
                pltpu.sync_copy(ys[k].at[pl.ds(sid * RPS, RPS)],
                                acc.at[pl.ds(sid * RPS, RPS)])

            @pl.when(cid != 0)
            def _():
                pltpu.sync_copy(zeros_hbm.at[pl.ds(sid * RPS, RPS)],
                                acc.at[pl.ds(sid * RPS, RPS)])

            plsc.subcore_barrier()

            def _gather(j, buf, sem, k=k):
                pltpu.async_copy(ys[k].at[srcbuf.at[j]], buf, sem)

            def _gwait(buf, sem, k=k):
                pltpu.make_async_copy(ys[k].at[srcbuf.at[0]], buf, sem).wait()

            _gather(0, rows_a, sem_a)

            @pl.loop(0, NB_W, step=2)
            def _(j, k=k):
                _gather(j + 1, rows_b, sem_b)
                _gwait(rows_a, sem_a)
                pltpu.sync_copy(rows_a, acc.at[dstbuf.at[j]], add=True)

                @pl.when(j + 2 < NB_W)
                def _():
                    _gather(j + 2, rows_a, sem_a)

                _gwait(rows_b, sem_b)
                pltpu.sync_copy(rows_b, acc.at[dstbuf.at[j + 1]], add=True)

            plsc.subcore_barrier()

            @pl.when(sid == 0)
            def _(k=k):
                pltpu.sync_copy(acc, outs[k].at[cid])

            if k < K - 1:
                plsc.subcore_barrier()

    return agg


_sc_agg2 = _make_sc_agg(2)
_sc_agg4 = _make_sc_agg(4)


def _prescale_body(deg_ref, x_ref, y0_ref, y1_ref, dinv_ref):
    dp = deg_ref[...]
    deg = dp[0, :, 0:1] + dp[1, :, 0:1] + 1.0
    dv = lax.rsqrt(deg)
    rid = jax.lax.broadcasted_iota(jnp.int32, (R, 1), 0) + pl.program_id(0) * R
    mask = rid < N
    dv = jnp.where(mask, dv, 0.0)
    xb = x_ref[...]
    y = jnp.where(mask, xb * dv, 0.0)
    y0_ref[...] = y[:, :C]
    y1_ref[...] = y[:, C:]
    dinv_ref[...] = jnp.broadcast_to(dv, (R, C))


def _tc_prescale(deg_parts, x):
    return pl.pallas_call(
        _prescale_body,
        grid=(GRID,),
        in_specs=[
            pl.BlockSpec((NC, R, C), lambda r: (0, r, 0)),
            pl.BlockSpec((R, 2 * C), lambda r: (r, 0)),
        ],
        out_specs=[
            pl.BlockSpec((R, C), lambda r: (r, 0)),
            pl.BlockSpec((R, C), lambda r: (r, 0)),
            pl.BlockSpec((R, C), lambda r: (r, 0)),
        ],
        out_shape=[
            jax.ShapeDtypeStruct((P, C), jnp.float32),
            jax.ShapeDtypeStruct((P, C), jnp.float32),
            jax.ShapeDtypeStruct((P, C), jnp.float32),
        ],
    )(deg_parts, x)


def _make_mm(K, F_out, final):
    def body(*refs):
        zs = refs[:K]
        dinv_ref, w_ref, b_ref, a_ref = refs[K:K + 4]
        outs = refs[K + 4:]
        dv = dinv_ref[:, 0:1]
        acc = b_ref[...]
        for k in range(K):
            zk = (zs[k][0] + zs[k][1]) * dv
            acc = acc + jnp.dot(zk, w_ref[k * C:(k + 1) * C, :],
                                preferred_element_type=jnp.float32)
        a = a_ref[0, 0]
        h = jnp.where(acc >= 0, acc, a * acc)
        if final:
            outs[0][...] = h
        else:
            h = h * dv
            for k in range(F_out // C):
                outs[k][...] = h[:, k * C:(k + 1) * C]

    n_out = 1 if final else F_out // C
    out_rows = N if final else P
    out_cols = F_out if final else C

    def run(zparts, dinv, W, b, a):
        return pl.pallas_call(
            body,
            grid=(GRID,),
            in_specs=(
                [pl.BlockSpec((NC, R, C), lambda r: (0, r, 0))
                 for _ in range(K)]
                + [
                    pl.BlockSpec((R, C), lambda r: (r, 0)),
                    pl.BlockSpec((K * C, F_out), lambda r: (0, 0)),
                    pl.BlockSpec((1, F_out), lambda r: (0, 0)),
                    pl.BlockSpec(memory_space=pltpu.SMEM),
                ]
            ),
            out_specs=[
                pl.BlockSpec((R, out_cols), lambda r: (r, 0))
                for _ in range(n_out)
            ],
            out_shape=[
                jax.ShapeDtypeStruct((out_rows, out_cols), jnp.float32)
                for _ in range(n_out)
            ],
        )(*zparts, dinv, W, b, a)

    return run


_tc_mm1 = _make_mm(2, 512, final=False)
_tc_mm2 = _make_mm(4, 512, final=True)


def kernel(x, edge_index, W1, b1, W2, b2, prelu_a):
    src2d = edge_index[0].reshape(NB, EB)
    dst2d = edge_index[1].reshape(NB, EB)
    onesC = jnp.ones((EB, C), jnp.float32)
    zerosC = jnp.zeros((P, C), jnp.float32)
    a2d = prelu_a.reshape(1, 1)
    b1r = b1.reshape(1, 512)
    b2r = b2.reshape(1, 512)

    deg_parts = _sc_deg(dst2d, onesC, zerosC)
    y10, y11, dinv = _tc_prescale(deg_parts, x)
    z1 = _sc_agg2(src2d, dst2d, zerosC, y10, y11)
    y2 = _tc_mm1(z1, dinv, W1, b1r, a2d)
    z2 = _sc_agg4(src2d, dst2d, zerosC, *y2)
    (out,) = _tc_mm2(z2, dinv, W2, b2r, a2d)
    return out


# speedup vs baseline: 17.5888x; 1.0137x over previous
import functools

import jax
import jax.numpy as jnp
from jax import lax
from jax.experimental import pallas as pl
from jax.experimental.pallas import tpu as pltpu
from jax.experimental.pallas import tpu_sc as plsc

N = 10000
E = 160000
P = 10240
C = 128
NC = 2
NS = 16
NW = NC * NS
EB = 125
NB = E // EB
NB_W = NB // NW
RPS = P // NS
R = 1024
GRID = P // R

_mesh = plsc.VectorSubcoreMesh(core_axis_name="c", subcore_axis_name="s")


@functools.partial(
    pl.kernel,
    out_type=jax.ShapeDtypeStruct((NC, P, C), jnp.float32),
    mesh=_mesh,
    scratch_types=[
        pltpu.VMEM_SHARED((P, C), jnp.float32),
        pltpu.VMEM((NB_W, EB), jnp.int32),
        pltpu.VMEM((EB, C), jnp.float32),
    ],
)
def _sc_deg(dst_hbm, ones_hbm, zeros_hbm, out_hbm, acc, dstbuf, ones):
    cid = lax.axis_index("c")
    sid = lax.axis_index("s")
    w = cid * NS + sid
    pltpu.sync_copy(ones_hbm, ones)
    pltpu.sync_copy(zeros_hbm.at[pl.ds(sid * RPS, RPS)],
                    acc.at[pl.ds(sid * RPS, RPS)])
    pltpu.sync_copy(dst_hbm.at[pl.ds(w * NB_W, NB_W)], dstbuf)
    plsc.subcore_barrier()

    @pl.loop(0, NB_W)
    def _(j):
        pltpu.sync_copy(ones, acc.at[dstbuf.at[j]], add=True)

    plsc.subcore_barrier()
    pltpu.sync_copy(acc.at[pl.ds(sid * RPS, RPS)],
                    out_hbm.at[cid].at[pl.ds(sid * RPS, RPS)])


def _make_sc_agg(K):
    @functools.partial(
        pl.kernel,
        out_type=[jax.ShapeDtypeStruct((NC, P, C), jnp.float32)
                  for _ in range(K)],
        mesh=_mesh,
        scratch_types=[
            pltpu.VMEM_SHARED((P, C), jnp.float32),
            pltpu.VMEM((NB_W, EB), jnp.int32),
            pltpu.VMEM((NB_W, EB), jnp.int32),
            pltpu.VMEM((EB, C), jnp.float32),
            pltpu.VMEM((EB, C), jnp.float32),
            pltpu.SemaphoreType.DMA,
            pltpu.SemaphoreType.DMA,
        ],
    )
    def agg(src_hbm, dst_hbm, zeros_hbm, *rest):
        ys = rest[:K]
        outs = rest[K:2 * K]
        acc, srcbuf, dstbuf, rows_a, rows_b, sem_a, sem_b = rest[2 * K:]
        cid = lax.axis_index("c")
        sid = lax.axis_index("s")
        w = cid * NS + sid
        pltpu.sync_copy(src_hbm.at[pl.ds(w * NB_W, NB_W)], srcbuf)
        pltpu.sync_copy(dst_hbm.at[pl.ds(w * NB_W, NB_W)], dstbuf)

        for k in range(K):
            @pl.when(cid == 0)
            def _(k=k):
                pltpu.sync_copy(ys[k].at[pl.ds(sid * RPS, RPS)],
                                acc.at[pl.ds(sid * RPS, RPS)])

            @pl.when(cid != 0)
            def _():
                pltpu.sync_copy(zeros_hbm.at[pl.ds(sid * RPS, RPS)],
                                acc.at[pl.ds(sid * RPS, RPS)])

            plsc.subcore_barrier()

            def _gather(j, buf, sem, k=k):
                pltpu.async_copy(ys[k].at[srcbuf.at[j]], buf, sem)

            def _gwait(buf, sem, k=k):
                pltpu.make_async_copy(ys[k].at[srcbuf.at[0]], buf, sem).wait()

            if k == 0:
                _gather(0, rows_a, sem_a)

            @pl.loop(0, NB_W, step=2)
            def _(j, k=k):
                _gather(j + 1, rows_b, sem_b)
                _gwait(rows_a, sem_a)
                pltpu.sync_copy(rows_a, acc.at[dstbuf.at[j]], add=True)

                @pl.when(j + 2 < NB_W)
                def _():
                    _gather(j + 2, rows_a, sem_a)

                _gwait(rows_b, sem_b)
                pltpu.sync_copy(rows_b, acc.at[dstbuf.at[j + 1]], add=True)

            if k < K - 1:
                pltpu.async_copy(ys[k + 1].at[srcbuf.at[0]], rows_a, sem_a)
            plsc.subcore_barrier()
            pltpu.sync_copy(acc.at[pl.ds(sid * RPS, RPS)],
                            outs[k].at[cid].at[pl.ds(sid * RPS, RPS)])

            if k < K - 1:
                plsc.subcore_barrier()

    return agg


_sc_agg2 = _make_sc_agg(2)
_sc_agg4 = _make_sc_agg(4)


def _prescale_body(deg_ref, x_ref, y0_ref, y1_ref, dinv_ref):
    dp = deg_ref[...]
    deg = dp[0, :, 0:1] + dp[1, :, 0:1] + 1.0
    dv = lax.rsqrt(deg)
    rid = jax.lax.broadcasted_iota(jnp.int32, (R, 1), 0) + pl.program_id(0) * R
    mask = rid < N
    dv = jnp.where(mask, dv, 0.0)
    xb = x_ref[...]
    y = jnp.where(mask, xb * dv, 0.0)
    y0_ref[...] = y[:, :C]
    y1_ref[...] = y[:, C:]
    dinv_ref[...] = jnp.broadcast_to(dv, (R, C))


def _tc_prescale(deg_parts, x):
    return pl.pallas_call(
        _prescale_body,
        grid=(GRID,),
        in_specs=[
            pl.BlockSpec((NC, R, C), lambda r: (0, r, 0)),
            pl.BlockSpec((R, 2 * C), lambda r: (r, 0)),
        ],
        out_specs=[
            pl.BlockSpec((R, C), lambda r: (r, 0)),
            pl.BlockSpec((R, C), lambda r: (r, 0)),
            pl.BlockSpec((R, C), lambda r: (r, 0)),
        ],
        out_shape=[
            jax.ShapeDtypeStruct((P, C), jnp.float32),
            jax.ShapeDtypeStruct((P, C), jnp.float32),
            jax.ShapeDtypeStruct((P, C), jnp.float32),
        ],
    )(deg_parts, x)


def _make_mm(K, F_out, final):
    def body(*refs):
        zs = refs[:K]
        dinv_ref, w_ref, b_ref, a_ref = refs[K:K + 4]
        outs = refs[K + 4:]
        dv = dinv_ref[:, 0:1]
        acc = b_ref[...]
        for k in range(K):
            zk = (zs[k][0] + zs[k][1]) * dv
            acc = acc + jnp.dot(zk, w_ref[k * C:(k + 1) * C, :],
                                preferred_element_type=jnp.float32)
        a = a_ref[0, 0]
        h = jnp.where(acc >= 0, acc, a * acc)
        if final:
            outs[0][...] = h
        else:
            h = h * dv
            for k in range(F_out // C):
                outs[k][...] = h[:, k * C:(k + 1) * C]

    n_out = 1 if final else F_out // C
    out_rows = N if final else P
    out_cols = F_out if final else C

    def run(zparts, dinv, W, b, a):
        return pl.pallas_call(
            body,
            grid=(GRID,),
            in_specs=(
                [pl.BlockSpec((NC, R, C), lambda r: (0, r, 0))
                 for _ in range(K)]
                + [
                    pl.BlockSpec((R, C), lambda r: (r, 0)),
                    pl.BlockSpec((K * C, F_out), lambda r: (0, 0)),
                    pl.BlockSpec((1, F_out), lambda r: (0, 0)),
                    pl.BlockSpec(memory_space=pltpu.SMEM),
                ]
            ),
            out_specs=[
                pl.BlockSpec((R, out_cols), lambda r: (r, 0))
                for _ in range(n_out)
            ],
            out_shape=[
                jax.ShapeDtypeStruct((out_rows, out_cols), jnp.float32)
                for _ in range(n_out)
            ],
        )(*zparts, dinv, W, b, a)

    return run


_tc_mm1 = _make_mm(2, 512, final=False)
_tc_mm2 = _make_mm(4, 512, final=True)


def kernel(x, edge_index, W1, b1, W2, b2, prelu_a):
    src2d = edge_index[0].reshape(NB, EB)
    dst2d = edge_index[1].reshape(NB, EB)
    onesC = jnp.ones((EB, C), jnp.float32)
    zerosC = jnp.zeros((P, C), jnp.float32)
    a2d = prelu_a.reshape(1, 1)
    b1r = b1.reshape(1, 512)
    b2r = b2.reshape(1, 512)

    deg_parts = _sc_deg(dst2d, onesC, zerosC)
    y10, y11, dinv = _tc_prescale(deg_parts, x)
    z1 = _sc_agg2(src2d, dst2d, zerosC, y10, y11)
    y2 = _tc_mm1(z1, dinv, W1, b1r, a2d)
    z2 = _sc_agg4(src2d, dst2d, zerosC, *y2)
    (out,) = _tc_mm2(z2, dinv, W2, b2r, a2d)
    return out
